# Initial kernel scaffold; baseline (speedup 1.0000x reference)
import jax
import jax.numpy as jnp
from jax.experimental import pallas as pl


def kernel(eu_emb, locations):
    raise NotImplementedError("write your pallas kernel here")


# speedup vs baseline: 591.1609x; 591.1609x over previous
import functools

import jax
import jax.numpy as jnp
import numpy as np
from jax.experimental import pallas as pl

_Z = np.int32(0)

jax.config.update("jax_enable_x64", True)

H = 8
TP = 2
CC = 2
N = 2 ** H
K = (CC * H) // 2
VOCAB = 512
BATCH = 128
D = 256
NCNC = 2 * H * K + 1
NC = 136


def _mm_kernel(a_ref, o_ref):
    a = a_ref[:, :]
    o_ref[:, :] = jax.lax.dot_general(
        a, a, (((1,), (1,)), ((), ())),
        precision=jax.lax.Precision.DEFAULT,
        preferred_element_type=jnp.float32)


def _hb(x):
    y = x + 1
    acc = jnp.zeros(x.shape, jnp.float32)
    for i in range(9):
        acc = acc + (y >= (1 << i)).astype(jnp.float32)
    return acc


def _step_kernel(loc_abs_ref, loc_sgn_ref, sta_abs_ref, sta_sgn_ref,
                 cnc_ref, logit_ref, sel_ref, tl_ref):
    logit = logit_ref[0, :, :]
    d_sp = []
    for t in range(TP):
        loc_a = loc_abs_ref[t:t + 1, :]
        loc_s = loc_sgn_ref[t:t + 1, :]
        sa = sta_abs_ref[0, 0, t]
        ss = sta_sgn_ref[0, 0, t]
        xr = jnp.bitwise_xor(sa, loc_a)
        d_sp.append((ss * loc_s) * (1.0 - _hb(xr) * 0.125))
    posum = d_sp[0] + d_sp[1]

    ridx = jax.lax.broadcasted_iota(jnp.int32, (NC, 1), 0)
    sels = []
    tls = []
    for t in range(TP):
        cnc_t = cnc_ref[0, :, t:t + 1]
        c_abs = jnp.abs(cnc_t)
        c_sgn = jnp.sign(cnc_t).astype(jnp.float32)
        loc_a = loc_abs_ref[t:t + 1, :]
        loc_s = loc_sgn_ref[t:t + 1, :]
        xr2 = jnp.bitwise_xor(c_abs, loc_a)
        d2 = (c_sgn * loc_s) * (1.0 - _hb(xr2) * 0.125)
        dn = ((d2 - d_sp[t]) + posum) * 0.5 - logit
        lossv = jnp.sum(jnp.abs(dn), axis=1, keepdims=True)
        lossv = lossv * (1.0 / VOCAB)
        lossm = jnp.where(ridx < NCNC, lossv, jnp.float32(1e30))
        m = jnp.min(lossm)
        idx = jnp.min(jnp.where(lossm == m, ridx, NC))
        big = jnp.int32(2 ** 30)
        sel = jnp.min(jnp.where(ridx == idx, cnc_t, big))
        sels.append(jnp.reshape(sel, (1, 1)))
        tls.append(jnp.reshape(m, (1, 1)))
    sel_ref[0, :, :] = jnp.concatenate(sels, axis=1)
    tl_ref[0, :, :] = jnp.concatenate(tls, axis=1)


def _batch_step(loc_abs_t, loc_sgn_t, sta_abs, sta_sgn, cnc, logits_rows):
    return pl.pallas_call(
        _step_kernel,
        grid=(BATCH,),
        in_specs=[
            pl.BlockSpec((TP, VOCAB), lambda b: (_Z, _Z)),
            pl.BlockSpec((TP, VOCAB), lambda b: (_Z, _Z)),
            pl.BlockSpec((1, 1, TP), lambda b: (b, _Z, _Z)),
            pl.BlockSpec((1, 1, TP), lambda b: (b, _Z, _Z)),
            pl.BlockSpec((1, NC, TP), lambda b: (b, _Z, _Z)),
            pl.BlockSpec((1, 1, VOCAB), lambda b: (b, _Z, _Z)),
        ],
        out_specs=[
            pl.BlockSpec((1, 1, TP), lambda b: (b, _Z, _Z)),
            pl.BlockSpec((1, 1, TP), lambda b: (b, _Z, _Z)),
        ],
        out_shape=[
            jax.ShapeDtypeStruct((BATCH, 1, TP), jnp.int32),
            jax.ShapeDtypeStruct((BATCH, 1, TP), jnp.float32),
        ],
    )(loc_abs_t, loc_sgn_t, sta_abs, sta_sgn, cnc, logits_rows)


@jax.jit
def _run(eu_emb, locations):
    logits = pl.pallas_call(
        _mm_kernel,
        out_shape=jax.ShapeDtypeStruct((VOCAB, VOCAB), jnp.float32),
    )(eu_emb)

    key = jax.random.key(42)
    perm = jax.random.permutation(jax.random.fold_in(key, 0), VOCAB)
    flip_masks = (2 ** jnp.arange(H, dtype=jnp.int64))[None, :, None]
    upper = (2 ** jnp.arange(H, dtype=jnp.int64)).reshape(-1, 1, 1, 1)
    n_batches = VOCAB // BATCH

    loc = locations.astype(jnp.int32)
    tl_sum = jnp.float32(0.0)
    for b in range(n_batches):
        sta_ind = perm[b * BATCH:(b + 1) * BATCH]
        kb = jax.random.fold_in(key, b + 1)
        km, kp = jax.random.split(kb)
        sta_loc = loc[sta_ind].astype(jnp.int64)
        ori = jnp.abs(sta_loc)
        flipped = jnp.bitwise_xor(ori[:, None, :], flip_masks)
        rnd = jax.random.randint(km, (H, BATCH, K, TP), 0, N, dtype=jnp.int64)
        masks = jnp.transpose(rnd % upper, (1, 0, 2, 3))
        result = jnp.bitwise_xor(flipped[:, :, None, :], masks)
        result = result.reshape(BATCH, H * K, TP)
        cnc = jnp.concatenate([result, ori[:, None, :], -result], axis=1)
        pidx = jax.random.permutation(kp, NCNC)
        cnc = cnc[:, pidx, :].astype(jnp.int32)
        cnc = jnp.pad(cnc, ((0, 0), (0, NC - NCNC), (0, 0)))

        loc_abs_t = jnp.abs(loc).T
        loc_sgn_t = jnp.sign(loc).astype(jnp.float32).T
        sta_abs = jnp.abs(sta_loc).astype(jnp.int32)[:, None, :]
        sta_sgn = jnp.sign(sta_loc).astype(jnp.float32)[:, None, :]
        logits_rows = logits[sta_ind][:, None, :]

        sel, tlm = _batch_step(loc_abs_t, loc_sgn_t, sta_abs, sta_sgn,
                               cnc, logits_rows)
        loc = loc.at[sta_ind].set(sel[:, 0, :])
        tl_sum = tl_sum + jnp.mean(tlm)
    return loc.astype(jnp.int64), tl_sum / n_batches


def kernel(eu_emb, locations):
    return _run(eu_emb, locations)
